# baseline (speedup 1.0000x reference)
import jax
import jax.numpy as jnp
from jax.experimental import pallas as pl
from jax.experimental.pallas import tpu as pltpu

_TOP_K = 8
_MASK_VAL = 1e8


def _loss_body(pred_ref, tgt_ref, w_ref, acc_ref):
    i = pl.program_id(0)
    p = pred_ref[...]
    g = tgt_ref[...]
    d = p - g
    s = jnp.sum(d * d, axis=3)
    s = jnp.sum(s, axis=2)

    w = jnp.sum(w_ref[...], axis=-1)
    hw = p.shape[2] * p.shape[3]
    loss = (0.5 / hw) * (w * w) * s
    key = jnp.where(w > 0.0, loss, _MASK_VAL)

    bb, j = key.shape
    cols = jax.lax.broadcasted_iota(jnp.int32, (bb, j), 1)
    tot = jnp.zeros((), jnp.float32)
    for _ in range(_TOP_K):
        m = jnp.min(key, axis=-1, keepdims=True)
        tot = tot + jnp.sum(jnp.where(m < _MASK_VAL, m, 0.0))
        cand = jnp.where(key == m, cols, j + 1)
        cmin = jnp.min(cand, axis=-1, keepdims=True)
        key = jnp.where(cols == cmin, jnp.float32(3e38), key)

    @pl.when(i == 0)
    def _():
        acc_ref[0, 0] = 0.0
    acc_ref[0, 0] += tot


def kernel(output, target, target_weight, top_k):
    batch, joints, h, w = output.shape
    bb = 8
    grid = (batch // bb,)
    acc = pl.pallas_call(
        _loss_body,
        grid=grid,
        in_specs=[
            pl.BlockSpec((bb, joints, h, w), lambda i: (i, 0, 0, 0)),
            pl.BlockSpec((bb, joints, h, w), lambda i: (i, 0, 0, 0)),
            pl.BlockSpec((bb, joints, 1), lambda i: (i, 0, 0)),
        ],
        out_specs=pl.BlockSpec(memory_space=pltpu.SMEM),
        out_shape=jax.ShapeDtypeStruct((1, 1), jnp.float32),
        compiler_params=pltpu.CompilerParams(
            dimension_semantics=("arbitrary",),
        ),
    )(output, target, target_weight)
    return acc[0, 0] / (top_k * batch)


# speedup vs baseline: 1.0345x; 1.0345x over previous
import jax
import jax.numpy as jnp
from jax.experimental import pallas as pl
from jax.experimental.pallas import tpu as pltpu

_TOP_K = 8
_MASK_VAL = 1e8


def _loss_body(pred_ref, tgt_ref, w_ref, acc_ref):
    i = pl.program_id(0)
    p = pred_ref[...]
    g = tgt_ref[...]
    d = p - g
    s = jnp.sum(d * d, axis=2)
    s = jnp.sum(s, axis=-1)

    w = jnp.sum(w_ref[...], axis=-1)
    hw = p.shape[2] * p.shape[3]
    loss = (0.5 / hw) * (w * w) * s
    key = jnp.where(w > 0.0, loss, _MASK_VAL)

    bb, j = key.shape
    cols = jax.lax.broadcasted_iota(jnp.int32, (bb, j), 1)
    tot = jnp.zeros((), jnp.float32)
    for _ in range(_TOP_K):
        m = jnp.min(key, axis=-1, keepdims=True)
        tot = tot + jnp.sum(jnp.where(m < _MASK_VAL, m, 0.0))
        cand = jnp.where(key == m, cols, j + 1)
        cmin = jnp.min(cand, axis=-1, keepdims=True)
        key = jnp.where(cols == cmin, jnp.float32(3e38), key)

    @pl.when(i == 0)
    def _():
        acc_ref[0, 0] = 0.0
    acc_ref[0, 0] += tot


def kernel(output, target, target_weight, top_k):
    batch, joints, h, w = output.shape
    bb = 8
    grid = (batch // bb,)
    acc = pl.pallas_call(
        _loss_body,
        grid=grid,
        in_specs=[
            pl.BlockSpec((bb, joints, h, w), lambda i: (i, 0, 0, 0)),
            pl.BlockSpec((bb, joints, h, w), lambda i: (i, 0, 0, 0)),
            pl.BlockSpec((bb, joints, 1), lambda i: (i, 0, 0)),
        ],
        out_specs=pl.BlockSpec(memory_space=pltpu.SMEM),
        out_shape=jax.ShapeDtypeStruct((1, 1), jnp.float32),
        compiler_params=pltpu.CompilerParams(
            dimension_semantics=("arbitrary",),
        ),
    )(output, target, target_weight)
    return acc[0, 0] / (top_k * batch)


# speedup vs baseline: 7.8197x; 7.5587x over previous
import jax
import jax.numpy as jnp
from jax.experimental import pallas as pl
from jax.experimental.pallas import tpu as pltpu

_TOP_K = 8
_MASK_VAL = 1e8


def _body(p_ref, g_ref, w_ref, out_ref, key_ref):
    j = pl.program_id(0)
    nj = pl.num_programs(0)
    p = p_ref[0]
    g = g_ref[0]
    d = p - g
    s = jnp.sum(d * d, axis=0)
    s = jnp.sum(s, axis=0, keepdims=True)

    w = w_ref[0]
    hw = p.shape[0] * p.shape[1]
    loss = (0.5 / hw) * (w * w) * s
    key_ref[pl.ds(j, 1), :] = jnp.where(w > 0.0, loss, _MASK_VAL)

    @pl.when(j == nj - 1)
    def _():
        key = key_ref[...]
        rows = jax.lax.broadcasted_iota(jnp.int32, key.shape, 0)
        tot = jnp.zeros((1, key.shape[1]), jnp.float32)
        for _ in range(_TOP_K):
            m = jnp.min(key, axis=0, keepdims=True)
            tot = tot + jnp.where(m < _MASK_VAL, m, 0.0)
            cand = jnp.where(key == m, rows, key.shape[0] + 1)
            rmin = jnp.min(cand, axis=0, keepdims=True)
            key = jnp.where(rows == rmin, jnp.float32(3e38), key)
        out_ref[0, 0] = jnp.sum(tot)


def kernel(output, target, target_weight, top_k):
    batch, joints, h, w = output.shape
    pt = jnp.transpose(output, (1, 2, 3, 0))
    gt = jnp.transpose(target, (1, 2, 3, 0))
    wt = jnp.transpose(target_weight, (1, 2, 0))
    acc = pl.pallas_call(
        _body,
        grid=(joints,),
        in_specs=[
            pl.BlockSpec((1, h, w, batch), lambda j: (j, 0, 0, 0)),
            pl.BlockSpec((1, h, w, batch), lambda j: (j, 0, 0, 0)),
            pl.BlockSpec((1, 1, batch), lambda j: (j, 0, 0)),
        ],
        out_specs=pl.BlockSpec(memory_space=pltpu.SMEM),
        out_shape=jax.ShapeDtypeStruct((1, 1), jnp.float32),
        scratch_shapes=[pltpu.VMEM((joints, batch), jnp.float32)],
        compiler_params=pltpu.CompilerParams(
            dimension_semantics=("arbitrary",),
        ),
    )(pt, gt, wt)
    return acc[0, 0] / (top_k * batch)
